# baseline (speedup 1.0000x reference)
import functools

import jax
import jax.numpy as jnp
from jax import lax
from jax.experimental import pallas as pl
from jax.experimental.pallas import tpu as pltpu
from jax.experimental.pallas import tpu_sc as plsc

N = 10000
NP = 10240
D = 128
NINC = 320000
NC, NS = 2, 16
NW = NC * NS
CHUNK = 64
INC_PER_W = 10240
NINC_PAD = NW * INC_PER_W
NCHUNKS = INC_PER_W // CHUNK
CPI = 16
ROWS_PER_TILE = NP // NS
ZCHUNKS = ROWS_PER_TILE // CHUNK
DEGW = 8

_mesh = plsc.VectorSubcoreMesh(
    core_axis_name="c", subcore_axis_name="s", num_cores=NC, num_subcores=NS)


def _scatter_body(with_deg, chunk, nbuf, cpi, *refs):
    nchunks = INC_PER_W // chunk
    zchunks = ROWS_PER_TILE // chunk
    if with_deg:
        (x_hbm, sidx_hbm, didx_hbm, z128_hbm, z18_hbm,
         part_out, deg_out,
         acc_sh, deg_sh,
         sa, da, *rest) = refs
        rows = rest[:nbuf]
        ones8_v = rest[nbuf]
        sg = rest[nbuf + 1:2 * nbuf + 1]
        ss = rest[2 * nbuf + 1:3 * nbuf + 1]
        sd, sz = rest[3 * nbuf + 1:]
    else:
        (x_hbm, sidx_hbm, didx_hbm, z128_hbm, z18_hbm,
         part_out,
         acc_sh,
         sa, da, *rest) = refs
        rows = rest[:nbuf]
        sg = rest[nbuf:2 * nbuf]
        ss = rest[2 * nbuf:3 * nbuf]
        (sz,) = rest[3 * nbuf:]
        deg_sh = ones8_v = sd = None
    c = lax.axis_index("c")
    s = lax.axis_index("s")
    r0 = s * ROWS_PER_TILE
    out_base = c * NP + r0
    rows0 = rows[0]

    pltpu.sync_copy(z128_hbm, rows0)
    for j in range(zchunks):
        pltpu.async_copy(rows0, acc_sh.at[pl.ds(r0 + j * chunk, chunk)], sz)
    for j in range(zchunks):
        pltpu.make_async_copy(
            rows0, acc_sh.at[pl.ds(r0 + j * chunk, chunk)], sz).wait()
    if with_deg:
        pltpu.sync_copy(z18_hbm.at[pl.ds(0, chunk)], ones8_v)
        for j in range(zchunks):
            pltpu.async_copy(ones8_v,
                             deg_sh.at[pl.ds(r0 + j * chunk, chunk)], sz)
        for j in range(zchunks):
            pltpu.make_async_copy(
                ones8_v, deg_sh.at[pl.ds(r0 + j * chunk, chunk)], sz).wait()
        pltpu.sync_copy(z18_hbm.at[pl.ds(128, chunk)], ones8_v)
    plsc.subcore_barrier()

    wid = c * NS + s
    idx_row_base = wid * (INC_PER_W // chunk)

    def super_body(t, carry):
        row0 = idx_row_base + t * cpi
        ia = pltpu.async_copy(sidx_hbm.at[pl.ds(row0, cpi)], sa, sz)
        ib = pltpu.async_copy(didx_hbm.at[pl.ds(row0, cpi)], da, sz)
        ia.wait()
        ib.wait()
        g_pend = [None] * nbuf
        s_pend = [None] * nbuf
        d_pend = [None]
        for p in range(nbuf - 1):
            g_pend[p] = pltpu.async_copy(x_hbm.at[sa.at[p]], rows[p], sg[p])
        for j in range(cpi):
            p = j % nbuf
            g_pend[p].wait()
            g_pend[p] = None
            s_pend[p] = pltpu.async_copy(
                rows[p], acc_sh.at[da.at[j]], ss[p], add=True)
            if with_deg:
                if d_pend[0] is not None:
                    d_pend[0].wait()
                d_pend[0] = pltpu.async_copy(
                    ones8_v, deg_sh.at[da.at[j]], sd, add=True)
            if j + nbuf - 1 < cpi:
                q = (j + nbuf - 1) % nbuf
                if s_pend[q] is not None:
                    s_pend[q].wait()
                    s_pend[q] = None
                g_pend[q] = pltpu.async_copy(
                    x_hbm.at[sa.at[j + nbuf - 1]], rows[q], sg[q])
        for p in range(nbuf):
            if s_pend[p] is not None:
                s_pend[p].wait()
        if with_deg and d_pend[0] is not None:
            d_pend[0].wait()
        return carry

    lax.fori_loop(0, nchunks // cpi, super_body, 0)
    plsc.subcore_barrier()

    for j in range(zchunks):
        p = j % 2
        if j >= 2:
            pltpu.make_async_copy(
                rows[p],
                part_out.at[pl.ds(out_base + (j - 2) * chunk, chunk)],
                ss[p]).wait()
        pltpu.sync_copy(acc_sh.at[pl.ds(r0 + j * chunk, chunk)], rows[p])
        pltpu.async_copy(rows[p],
                         part_out.at[pl.ds(out_base + j * chunk, chunk)],
                         ss[p])
    for j in (zchunks - 2, zchunks - 1):
        pltpu.make_async_copy(
            rows[j % 2], part_out.at[pl.ds(out_base + j * chunk, chunk)],
            ss[j % 2]).wait()
    if with_deg:
        for j in range(zchunks):
            pltpu.sync_copy(deg_sh.at[pl.ds(r0 + j * chunk, chunk)], ones8_v)
            pltpu.sync_copy(ones8_v,
                            deg_out.at[pl.ds(out_base + j * chunk, chunk)])


def _make_scatter(with_deg, chunk, nbuf, cpi):
    out_type = [jax.ShapeDtypeStruct((NC * NP, D), jnp.float32)]
    scratch = [
        pltpu.VMEM_SHARED((NP, D), jnp.float32),
    ]
    if with_deg:
        out_type.append(jax.ShapeDtypeStruct((NC * NP, DEGW), jnp.float32))
        scratch.append(pltpu.VMEM_SHARED((NP, DEGW), jnp.float32))
    scratch += [
        pltpu.VMEM((cpi, chunk), jnp.int32),
        pltpu.VMEM((cpi, chunk), jnp.int32),
    ]
    scratch += [pltpu.VMEM((chunk, D), jnp.float32)] * nbuf
    if with_deg:
        scratch.append(pltpu.VMEM((chunk, DEGW), jnp.float32))
    nsem = 2 * nbuf + (2 if with_deg else 1)
    scratch += [pltpu.SemaphoreType.DMA] * nsem
    return pl.kernel(
        functools.partial(_scatter_body, with_deg, chunk, nbuf, cpi),
        out_type=tuple(out_type),
        mesh=_mesh,
        scratch_types=tuple(scratch),
        compiler_params=pltpu.CompilerParams(use_tc_tiling_on_sc=False),
    )


DEG_CHUNK = 64
PLAIN_CHUNK = 128
_scatter_deg = _make_scatter(True, DEG_CHUNK, 3, 16)
_scatter_plain = _make_scatter(False, PLAIN_CHUNK, 2, 8)


_BLK = 512
_GRID = NP // _BLK


def _mm_body(x_ref, w_ref, o_ref):
    o_ref[...] = jnp.dot(x_ref[...], w_ref[...],
                         preferred_element_type=jnp.float32)


_mm = pl.pallas_call(
    _mm_body,
    grid=(_GRID,),
    in_specs=[
        pl.BlockSpec((_BLK, D), lambda i: (i, 0)),
        pl.BlockSpec((D, D), lambda i: (0, 0)),
    ],
    out_specs=pl.BlockSpec((_BLK, D), lambda i: (i, 0)),
    out_shape=jax.ShapeDtypeStruct((NP, D), jnp.float32),
)


def _recip_pos(x):
    return jnp.where(x > 0, 1.0 / jnp.where(x > 0, x, 1.0), 0.0)


def _combine_scale_body(pe_ref, bp_ref, o_ref):
    ssum = pe_ref[0] + pe_ref[1]
    cnt = bp_ref[0, :, 0:1] + bp_ref[1, :, 0:1]
    o_ref[...] = ssum * _recip_pos(cnt)


_combine_scale = pl.pallas_call(
    _combine_scale_body,
    grid=(_GRID,),
    in_specs=[
        pl.BlockSpec((NC, _BLK, D), lambda i: (0, i, 0)),
        pl.BlockSpec((NC, _BLK, DEGW), lambda i: (0, i, 0)),
    ],
    out_specs=pl.BlockSpec((_BLK, D), lambda i: (i, 0)),
    out_shape=jax.ShapeDtypeStruct((NP, D), jnp.float32),
)


def _lrelu(x):
    return jnp.where(x >= 0, x, 0.01 * x)


def _combine_relu_mm_body(pn_ref, dp_ref, b_ref, w_ref, o_ref):
    ssum = pn_ref[0] + pn_ref[1]
    cnt = dp_ref[0, :, 0:1] + dp_ref[1, :, 0:1]
    h = _lrelu(ssum * _recip_pos(cnt) + b_ref[...])
    o_ref[...] = jnp.dot(h, w_ref[...], preferred_element_type=jnp.float32)


_combine_relu_mm = pl.pallas_call(
    _combine_relu_mm_body,
    grid=(_GRID,),
    in_specs=[
        pl.BlockSpec((NC, _BLK, D), lambda i: (0, i, 0)),
        pl.BlockSpec((NC, _BLK, DEGW), lambda i: (0, i, 0)),
        pl.BlockSpec((1, D), lambda i: (0, 0)),
        pl.BlockSpec((D, D), lambda i: (0, 0)),
    ],
    out_specs=pl.BlockSpec((_BLK, D), lambda i: (i, 0)),
    out_shape=jax.ShapeDtypeStruct((NP, D), jnp.float32),
)


def _combine_relu_body(pn_ref, dp_ref, b_ref, o_ref):
    ssum = pn_ref[0] + pn_ref[1]
    cnt = dp_ref[0, :, 0:1] + dp_ref[1, :, 0:1]
    o_ref[...] = _lrelu(ssum * _recip_pos(cnt) + b_ref[...])


_combine_relu = pl.pallas_call(
    _combine_relu_body,
    grid=(_GRID,),
    in_specs=[
        pl.BlockSpec((NC, _BLK, D), lambda i: (0, i, 0)),
        pl.BlockSpec((NC, _BLK, DEGW), lambda i: (0, i, 0)),
        pl.BlockSpec((1, D), lambda i: (0, 0)),
    ],
    out_specs=pl.BlockSpec((_BLK, D), lambda i: (i, 0)),
    out_shape=jax.ShapeDtypeStruct((NP, D), jnp.float32),
)


@jax.jit
def kernel(nodes_features, hyperedge_index, W1, b1, W2, b2):
    row = hyperedge_index[0].astype(jnp.int32)
    col = hyperedge_index[1].astype(jnp.int32)
    npad = NINC_PAD - NINC
    ar = jnp.arange(npad, dtype=jnp.int32)
    pad_g = (ar * 97) % N
    pad_s = N + ar % (NP - N)
    shp_d = (NINC_PAD // DEG_CHUNK, DEG_CHUNK)
    shp_p = (NINC_PAD // PLAIN_CHUNK, PLAIN_CHUNK)
    row_g = jnp.concatenate([row, pad_g])
    row_s = jnp.concatenate([row, pad_s])
    col_g = jnp.concatenate([col, pad_g])
    col_s = jnp.concatenate([col, pad_s])

    x_pad = jnp.zeros((NP, D), jnp.float32).at[:N].set(nodes_features)
    z64 = jnp.zeros((DEG_CHUNK, D), jnp.float32)
    z128p = jnp.zeros((PLAIN_CHUNK, D), jnp.float32)
    z18 = jnp.concatenate([jnp.zeros((128, DEGW), jnp.float32),
                           jnp.ones((128, DEGW), jnp.float32)])
    b1r = b1.reshape(1, D)
    b2r = b2.reshape(1, D)

    def _r(p):
        return p.reshape(NC, NP, p.shape[-1])

    xp1 = _mm(x_pad, W1)
    pe, bp = _scatter_deg(xp1, row_g.reshape(shp_d), col_s.reshape(shp_d),
                          z64, z18)
    ef = _combine_scale(_r(pe), _r(bp))
    pn, dp = _scatter_deg(ef, col_g.reshape(shp_d), row_s.reshape(shp_d),
                          z64, z18)
    xp2 = _combine_relu_mm(_r(pn), _r(dp), b1r, W2)
    (pe2,) = _scatter_plain(xp2, row_g.reshape(shp_p), col_s.reshape(shp_p),
                            z128p, z18)
    ef2 = _combine_scale(_r(pe2), _r(bp))
    (pn2,) = _scatter_plain(ef2, col_g.reshape(shp_p), row_s.reshape(shp_p),
                            z128p, z18)
    out = _combine_relu(_r(pn2), _r(dp), b2r)
    return out[:N]


# speedup vs baseline: 1.1042x; 1.1042x over previous
import functools

import jax
import jax.numpy as jnp
from jax import lax
from jax.experimental import pallas as pl
from jax.experimental.pallas import tpu as pltpu
from jax.experimental.pallas import tpu_sc as plsc

N = 10000
NP = 10240
D = 128
NINC = 320000
NC, NS = 2, 16
NW = NC * NS
CHUNK = 64
INC_PER_W = 10240
NINC_PAD = NW * INC_PER_W
NCHUNKS = INC_PER_W // CHUNK
CPI = 16
ROWS_PER_TILE = NP // NS
ZCHUNKS = ROWS_PER_TILE // CHUNK
DEGW = 8

_mesh = plsc.VectorSubcoreMesh(
    core_axis_name="c", subcore_axis_name="s", num_cores=NC, num_subcores=NS)


def _scatter_body(with_deg, chunk, nbuf, cpi, *refs):
    nchunks = INC_PER_W // chunk
    zchunks = ROWS_PER_TILE // chunk
    if with_deg:
        (x_hbm, sidx_hbm, didx_hbm, z128_hbm, z18_hbm,
         part_out, deg_out,
         acc_sh, deg_sh,
         sa, da, *rest) = refs
        rows = rest[:nbuf]
        ones8_v = rest[nbuf]
        sg = rest[nbuf + 1:2 * nbuf + 1]
        ss = rest[2 * nbuf + 1:3 * nbuf + 1]
        sd, sz = rest[3 * nbuf + 1:]
    else:
        (x_hbm, sidx_hbm, didx_hbm, z128_hbm, z18_hbm,
         part_out,
         acc_sh,
         sa, da, *rest) = refs
        rows = rest[:nbuf]
        sg = rest[nbuf:2 * nbuf]
        ss = rest[2 * nbuf:3 * nbuf]
        (sz,) = rest[3 * nbuf:]
        deg_sh = ones8_v = sd = None
    c = lax.axis_index("c")
    s = lax.axis_index("s")
    r0 = s * ROWS_PER_TILE
    out_base = c * NP + r0
    rows0 = rows[0]

    pltpu.sync_copy(z128_hbm, rows0)
    for j in range(zchunks):
        pltpu.async_copy(rows0, acc_sh.at[pl.ds(r0 + j * chunk, chunk)], sz)
    for j in range(zchunks):
        pltpu.make_async_copy(
            rows0, acc_sh.at[pl.ds(r0 + j * chunk, chunk)], sz).wait()
    if with_deg:
        pltpu.sync_copy(z18_hbm.at[pl.ds(0, chunk)], ones8_v)
        for j in range(zchunks):
            pltpu.async_copy(ones8_v,
                             deg_sh.at[pl.ds(r0 + j * chunk, chunk)], sz)
        for j in range(zchunks):
            pltpu.make_async_copy(
                ones8_v, deg_sh.at[pl.ds(r0 + j * chunk, chunk)], sz).wait()
        pltpu.sync_copy(z18_hbm.at[pl.ds(128, chunk)], ones8_v)
    plsc.subcore_barrier()

    wid = c * NS + s
    idx_row_base = wid * (INC_PER_W // chunk)

    def super_body(t, carry):
        row0 = idx_row_base + t * cpi
        ia = pltpu.async_copy(sidx_hbm.at[pl.ds(row0, cpi)], sa, sz)
        ib = pltpu.async_copy(didx_hbm.at[pl.ds(row0, cpi)], da, sz)
        ia.wait()
        ib.wait()
        g_pend = [None] * nbuf
        s_pend = [None] * nbuf
        d_pend = [None]
        for p in range(nbuf - 1):
            g_pend[p] = pltpu.async_copy(x_hbm.at[sa.at[p]], rows[p], sg[p])
        for j in range(cpi):
            p = j % nbuf
            g_pend[p].wait()
            g_pend[p] = None
            s_pend[p] = pltpu.async_copy(
                rows[p], acc_sh.at[da.at[j]], ss[p], add=True)
            if with_deg:
                if d_pend[0] is not None:
                    d_pend[0].wait()
                d_pend[0] = pltpu.async_copy(
                    ones8_v, deg_sh.at[da.at[j]], sd, add=True)
            if j + nbuf - 1 < cpi:
                q = (j + nbuf - 1) % nbuf
                if s_pend[q] is not None:
                    s_pend[q].wait()
                    s_pend[q] = None
                g_pend[q] = pltpu.async_copy(
                    x_hbm.at[sa.at[j + nbuf - 1]], rows[q], sg[q])
        for p in range(nbuf):
            if s_pend[p] is not None:
                s_pend[p].wait()
        if with_deg and d_pend[0] is not None:
            d_pend[0].wait()
        return carry

    lax.fori_loop(0, nchunks // cpi, super_body, 0)
    plsc.subcore_barrier()

    for j in range(zchunks):
        p = j % 2
        if j >= 2:
            pltpu.make_async_copy(
                rows[p],
                part_out.at[pl.ds(out_base + (j - 2) * chunk, chunk)],
                ss[p]).wait()
        pltpu.sync_copy(acc_sh.at[pl.ds(r0 + j * chunk, chunk)], rows[p])
        pltpu.async_copy(rows[p],
                         part_out.at[pl.ds(out_base + j * chunk, chunk)],
                         ss[p])
    for j in (zchunks - 2, zchunks - 1):
        pltpu.make_async_copy(
            rows[j % 2], part_out.at[pl.ds(out_base + j * chunk, chunk)],
            ss[j % 2]).wait()
    if with_deg:
        for j in range(zchunks):
            pltpu.sync_copy(deg_sh.at[pl.ds(r0 + j * chunk, chunk)], ones8_v)
            pltpu.sync_copy(ones8_v,
                            deg_out.at[pl.ds(out_base + j * chunk, chunk)])


def _make_scatter(with_deg, chunk, nbuf, cpi):
    out_type = [jax.ShapeDtypeStruct((NC * NP, D), jnp.float32)]
    scratch = [
        pltpu.VMEM_SHARED((NP, D), jnp.float32),
    ]
    if with_deg:
        out_type.append(jax.ShapeDtypeStruct((NC * NP, DEGW), jnp.float32))
        scratch.append(pltpu.VMEM_SHARED((NP, DEGW), jnp.float32))
    scratch += [
        pltpu.VMEM((cpi, chunk), jnp.int32),
        pltpu.VMEM((cpi, chunk), jnp.int32),
    ]
    scratch += [pltpu.VMEM((chunk, D), jnp.float32)] * nbuf
    if with_deg:
        scratch.append(pltpu.VMEM((chunk, DEGW), jnp.float32))
    nsem = 2 * nbuf + (2 if with_deg else 1)
    scratch += [pltpu.SemaphoreType.DMA] * nsem
    return pl.kernel(
        functools.partial(_scatter_body, with_deg, chunk, nbuf, cpi),
        out_type=tuple(out_type),
        mesh=_mesh,
        scratch_types=tuple(scratch),
        compiler_params=pltpu.CompilerParams(use_tc_tiling_on_sc=False),
    )


DEG_CHUNK = 64
PLAIN_CHUNK = 64
_scatter_deg = _make_scatter(True, DEG_CHUNK, 3, 20)
_scatter_plain = _make_scatter(False, PLAIN_CHUNK, 4, 32)


_BLK = 512
_GRID = NP // _BLK


def _mm_body(x_ref, w_ref, o_ref):
    o_ref[...] = jnp.dot(x_ref[...], w_ref[...],
                         preferred_element_type=jnp.float32)


_mm = pl.pallas_call(
    _mm_body,
    grid=(_GRID,),
    in_specs=[
        pl.BlockSpec((_BLK, D), lambda i: (i, 0)),
        pl.BlockSpec((D, D), lambda i: (0, 0)),
    ],
    out_specs=pl.BlockSpec((_BLK, D), lambda i: (i, 0)),
    out_shape=jax.ShapeDtypeStruct((NP, D), jnp.float32),
)


def _recip_pos(x):
    return jnp.where(x > 0, 1.0 / jnp.where(x > 0, x, 1.0), 0.0)


def _combine_scale_body(pe_ref, bp_ref, o_ref):
    ssum = pe_ref[0] + pe_ref[1]
    cnt = bp_ref[0, :, 0:1] + bp_ref[1, :, 0:1]
    o_ref[...] = ssum * _recip_pos(cnt)


_combine_scale = pl.pallas_call(
    _combine_scale_body,
    grid=(_GRID,),
    in_specs=[
        pl.BlockSpec((NC, _BLK, D), lambda i: (0, i, 0)),
        pl.BlockSpec((NC, _BLK, DEGW), lambda i: (0, i, 0)),
    ],
    out_specs=pl.BlockSpec((_BLK, D), lambda i: (i, 0)),
    out_shape=jax.ShapeDtypeStruct((NP, D), jnp.float32),
)


def _lrelu(x):
    return jnp.where(x >= 0, x, 0.01 * x)


def _combine_relu_mm_body(pn_ref, dp_ref, b_ref, w_ref, o_ref):
    ssum = pn_ref[0] + pn_ref[1]
    cnt = dp_ref[0, :, 0:1] + dp_ref[1, :, 0:1]
    h = _lrelu(ssum * _recip_pos(cnt) + b_ref[...])
    o_ref[...] = jnp.dot(h, w_ref[...], preferred_element_type=jnp.float32)


_combine_relu_mm = pl.pallas_call(
    _combine_relu_mm_body,
    grid=(_GRID,),
    in_specs=[
        pl.BlockSpec((NC, _BLK, D), lambda i: (0, i, 0)),
        pl.BlockSpec((NC, _BLK, DEGW), lambda i: (0, i, 0)),
        pl.BlockSpec((1, D), lambda i: (0, 0)),
        pl.BlockSpec((D, D), lambda i: (0, 0)),
    ],
    out_specs=pl.BlockSpec((_BLK, D), lambda i: (i, 0)),
    out_shape=jax.ShapeDtypeStruct((NP, D), jnp.float32),
)


def _combine_relu_body(pn_ref, dp_ref, b_ref, o_ref):
    ssum = pn_ref[0] + pn_ref[1]
    cnt = dp_ref[0, :, 0:1] + dp_ref[1, :, 0:1]
    o_ref[...] = _lrelu(ssum * _recip_pos(cnt) + b_ref[...])


_combine_relu = pl.pallas_call(
    _combine_relu_body,
    grid=(_GRID,),
    in_specs=[
        pl.BlockSpec((NC, _BLK, D), lambda i: (0, i, 0)),
        pl.BlockSpec((NC, _BLK, DEGW), lambda i: (0, i, 0)),
        pl.BlockSpec((1, D), lambda i: (0, 0)),
    ],
    out_specs=pl.BlockSpec((_BLK, D), lambda i: (i, 0)),
    out_shape=jax.ShapeDtypeStruct((NP, D), jnp.float32),
)


@jax.jit
def kernel(nodes_features, hyperedge_index, W1, b1, W2, b2):
    row = hyperedge_index[0].astype(jnp.int32)
    col = hyperedge_index[1].astype(jnp.int32)
    npad = NINC_PAD - NINC
    ar = jnp.arange(npad, dtype=jnp.int32)
    pad_g = (ar * 97) % N
    pad_s = N + ar % (NP - N)
    shp_d = (NINC_PAD // DEG_CHUNK, DEG_CHUNK)
    shp_p = (NINC_PAD // PLAIN_CHUNK, PLAIN_CHUNK)
    row_g = jnp.concatenate([row, pad_g])
    row_s = jnp.concatenate([row, pad_s])
    col_g = jnp.concatenate([col, pad_g])
    col_s = jnp.concatenate([col, pad_s])

    x_pad = jnp.zeros((NP, D), jnp.float32).at[:N].set(nodes_features)
    z64 = jnp.zeros((DEG_CHUNK, D), jnp.float32)
    z128p = jnp.zeros((PLAIN_CHUNK, D), jnp.float32)
    z18 = jnp.concatenate([jnp.zeros((128, DEGW), jnp.float32),
                           jnp.ones((128, DEGW), jnp.float32)])
    b1r = b1.reshape(1, D)
    b2r = b2.reshape(1, D)

    def _r(p):
        return p.reshape(NC, NP, p.shape[-1])

    xp1 = _mm(x_pad, W1)
    pe, bp = _scatter_deg(xp1, row_g.reshape(shp_d), col_s.reshape(shp_d),
                          z64, z18)
    ef = _combine_scale(_r(pe), _r(bp))
    pn, dp = _scatter_deg(ef, col_g.reshape(shp_d), row_s.reshape(shp_d),
                          z64, z18)
    xp2 = _combine_relu_mm(_r(pn), _r(dp), b1r, W2)
    (pe2,) = _scatter_plain(xp2, row_g.reshape(shp_p), col_s.reshape(shp_p),
                            z128p, z18)
    ef2 = _combine_scale(_r(pe2), _r(bp))
    (pn2,) = _scatter_plain(ef2, col_g.reshape(shp_p), row_s.reshape(shp_p),
                            z128p, z18)
    out = _combine_relu(_r(pn2), _r(dp), b2r)
    return out[:N]


# speedup vs baseline: 1.1121x; 1.0072x over previous
import functools

import jax
import jax.numpy as jnp
from jax import lax
from jax.experimental import pallas as pl
from jax.experimental.pallas import tpu as pltpu
from jax.experimental.pallas import tpu_sc as plsc

N = 10000
NP = 10240
D = 128
NINC = 320000
NC, NS = 2, 16
NW = NC * NS
CHUNK = 64
INC_PER_W = 10240
NINC_PAD = NW * INC_PER_W
NCHUNKS = INC_PER_W // CHUNK
CPI = 16
ROWS_PER_TILE = NP // NS
ZCHUNKS = ROWS_PER_TILE // CHUNK
DEGW = 8

_mesh = plsc.VectorSubcoreMesh(
    core_axis_name="c", subcore_axis_name="s", num_cores=NC, num_subcores=NS)


def _scatter_body(with_deg, chunk, nbuf, cpi, *refs):
    nchunks = INC_PER_W // chunk
    zchunks = ROWS_PER_TILE // chunk
    if with_deg:
        (x_hbm, sidx_hbm, didx_hbm, z128_hbm, z18_hbm,
         part_out, deg_out,
         acc_sh, deg_sh,
         sa, da, *rest) = refs
        rows = rest[:nbuf]
        ones8_v = rest[nbuf]
        sg = rest[nbuf + 1:2 * nbuf + 1]
        ss = rest[2 * nbuf + 1:3 * nbuf + 1]
        sd, sz = rest[3 * nbuf + 1:]
    else:
        (x_hbm, sidx_hbm, didx_hbm, z128_hbm, z18_hbm,
         part_out,
         acc_sh,
         sa, da, *rest) = refs
        rows = rest[:nbuf]
        sg = rest[nbuf:2 * nbuf]
        ss = rest[2 * nbuf:3 * nbuf]
        (sz,) = rest[3 * nbuf:]
        deg_sh = ones8_v = sd = None
    c = lax.axis_index("c")
    s = lax.axis_index("s")
    r0 = s * ROWS_PER_TILE
    out_base = c * NP + r0
    rows0 = rows[0]

    pltpu.sync_copy(z128_hbm, rows0)
    for j in range(zchunks):
        pltpu.async_copy(rows0, acc_sh.at[pl.ds(r0 + j * chunk, chunk)], sz)
    for j in range(zchunks):
        pltpu.make_async_copy(
            rows0, acc_sh.at[pl.ds(r0 + j * chunk, chunk)], sz).wait()
    if with_deg:
        pltpu.sync_copy(z18_hbm.at[pl.ds(0, chunk)], ones8_v)
        for j in range(zchunks):
            pltpu.async_copy(ones8_v,
                             deg_sh.at[pl.ds(r0 + j * chunk, chunk)], sz)
        for j in range(zchunks):
            pltpu.make_async_copy(
                ones8_v, deg_sh.at[pl.ds(r0 + j * chunk, chunk)], sz).wait()
        pltpu.sync_copy(z18_hbm.at[pl.ds(128, chunk)], ones8_v)
    plsc.subcore_barrier()

    wid = c * NS + s
    idx_row_base = wid * (INC_PER_W // chunk)

    def super_body(t, carry):
        row0 = idx_row_base + t * cpi
        ia = pltpu.async_copy(sidx_hbm.at[pl.ds(row0, cpi)], sa, sz)
        ib = pltpu.async_copy(didx_hbm.at[pl.ds(row0, cpi)], da, sz)
        ia.wait()
        ib.wait()
        g_pend = [None] * nbuf
        s_pend = [None] * nbuf
        d_pend = [None]
        for p in range(nbuf - 1):
            g_pend[p] = pltpu.async_copy(x_hbm.at[sa.at[p]], rows[p], sg[p])
        for j in range(cpi):
            p = j % nbuf
            g_pend[p].wait()
            g_pend[p] = None
            s_pend[p] = pltpu.async_copy(
                rows[p], acc_sh.at[da.at[j]], ss[p], add=True)
            if with_deg:
                if d_pend[0] is not None:
                    d_pend[0].wait()
                d_pend[0] = pltpu.async_copy(
                    ones8_v, deg_sh.at[da.at[j]], sd, add=True)
            if j + nbuf - 1 < cpi:
                q = (j + nbuf - 1) % nbuf
                if s_pend[q] is not None:
                    s_pend[q].wait()
                    s_pend[q] = None
                g_pend[q] = pltpu.async_copy(
                    x_hbm.at[sa.at[j + nbuf - 1]], rows[q], sg[q])
        for p in range(nbuf):
            if s_pend[p] is not None:
                s_pend[p].wait()
        if with_deg and d_pend[0] is not None:
            d_pend[0].wait()
        return carry

    lax.fori_loop(0, nchunks // cpi, super_body, 0)
    plsc.subcore_barrier()

    for j in range(zchunks):
        p = j % 2
        if j >= 2:
            pltpu.make_async_copy(
                rows[p],
                part_out.at[pl.ds(out_base + (j - 2) * chunk, chunk)],
                ss[p]).wait()
        pltpu.sync_copy(acc_sh.at[pl.ds(r0 + j * chunk, chunk)], rows[p])
        pltpu.async_copy(rows[p],
                         part_out.at[pl.ds(out_base + j * chunk, chunk)],
                         ss[p])
    for j in (zchunks - 2, zchunks - 1):
        pltpu.make_async_copy(
            rows[j % 2], part_out.at[pl.ds(out_base + j * chunk, chunk)],
            ss[j % 2]).wait()
    if with_deg:
        for j in range(zchunks):
            pltpu.sync_copy(deg_sh.at[pl.ds(r0 + j * chunk, chunk)], ones8_v)
            pltpu.sync_copy(ones8_v,
                            deg_out.at[pl.ds(out_base + j * chunk, chunk)])


def _make_scatter(with_deg, chunk, nbuf, cpi):
    out_type = [jax.ShapeDtypeStruct((NC * NP, D), jnp.float32)]
    scratch = [
        pltpu.VMEM_SHARED((NP, D), jnp.float32),
    ]
    if with_deg:
        out_type.append(jax.ShapeDtypeStruct((NC * NP, DEGW), jnp.float32))
        scratch.append(pltpu.VMEM_SHARED((NP, DEGW), jnp.float32))
    scratch += [
        pltpu.VMEM((cpi, chunk), jnp.int32),
        pltpu.VMEM((cpi, chunk), jnp.int32),
    ]
    scratch += [pltpu.VMEM((chunk, D), jnp.float32)] * nbuf
    if with_deg:
        scratch.append(pltpu.VMEM((chunk, DEGW), jnp.float32))
    nsem = 2 * nbuf + (2 if with_deg else 1)
    scratch += [pltpu.SemaphoreType.DMA] * nsem
    return pl.kernel(
        functools.partial(_scatter_body, with_deg, chunk, nbuf, cpi),
        out_type=tuple(out_type),
        mesh=_mesh,
        scratch_types=tuple(scratch),
        compiler_params=pltpu.CompilerParams(use_tc_tiling_on_sc=False),
    )


DEG_CHUNK = 64
PLAIN_CHUNK = 64
_scatter_deg = _make_scatter(True, DEG_CHUNK, 3, 20)
_scatter_plain = _make_scatter(False, PLAIN_CHUNK, 4, 32)


_BLK = 512
_GRID = NP // _BLK


def _mm_body(x_ref, w_ref, o_ref):
    o_ref[...] = jnp.dot(x_ref[...], w_ref[...],
                         preferred_element_type=jnp.float32)


_mm = pl.pallas_call(
    _mm_body,
    grid=(_GRID,),
    in_specs=[
        pl.BlockSpec((_BLK, D), lambda i: (i, 0)),
        pl.BlockSpec((D, D), lambda i: (0, 0)),
    ],
    out_specs=pl.BlockSpec((_BLK, D), lambda i: (i, 0)),
    out_shape=jax.ShapeDtypeStruct((NP, D), jnp.float32),
)


def _recip_pos(x):
    return jnp.where(x > 0, 1.0 / jnp.where(x > 0, x, 1.0), 0.0)


def _combine_scale_body(pe_ref, bp_ref, o_ref):
    ssum = pe_ref[0] + pe_ref[1]
    cnt = bp_ref[0, :, 0:1] + bp_ref[1, :, 0:1]
    o_ref[...] = ssum * _recip_pos(cnt)


_combine_scale = pl.pallas_call(
    _combine_scale_body,
    grid=(_GRID,),
    in_specs=[
        pl.BlockSpec((NC, _BLK, D), lambda i: (0, i, 0)),
        pl.BlockSpec((NC, _BLK, DEGW), lambda i: (0, i, 0)),
    ],
    out_specs=pl.BlockSpec((_BLK, D), lambda i: (i, 0)),
    out_shape=jax.ShapeDtypeStruct((NP, D), jnp.float32),
)


def _lrelu(x):
    return jnp.where(x >= 0, x, 0.01 * x)


def _combine_relu_mm_body(pn_ref, dp_ref, b_ref, w_ref, o_ref):
    ssum = pn_ref[0] + pn_ref[1]
    cnt = dp_ref[0, :, 0:1] + dp_ref[1, :, 0:1]
    h = _lrelu(ssum * _recip_pos(cnt) + b_ref[...])
    o_ref[...] = jnp.dot(h, w_ref[...], preferred_element_type=jnp.float32)


_combine_relu_mm = pl.pallas_call(
    _combine_relu_mm_body,
    grid=(_GRID,),
    in_specs=[
        pl.BlockSpec((NC, _BLK, D), lambda i: (0, i, 0)),
        pl.BlockSpec((NC, _BLK, DEGW), lambda i: (0, i, 0)),
        pl.BlockSpec((1, D), lambda i: (0, 0)),
        pl.BlockSpec((D, D), lambda i: (0, 0)),
    ],
    out_specs=pl.BlockSpec((_BLK, D), lambda i: (i, 0)),
    out_shape=jax.ShapeDtypeStruct((NP, D), jnp.float32),
)


def _combine_relu_body(pn_ref, dp_ref, b_ref, o_ref):
    ssum = pn_ref[0] + pn_ref[1]
    cnt = dp_ref[0, :, 0:1] + dp_ref[1, :, 0:1]
    o_ref[...] = _lrelu(ssum * _recip_pos(cnt) + b_ref[...])


_combine_relu = pl.pallas_call(
    _combine_relu_body,
    grid=(_GRID,),
    in_specs=[
        pl.BlockSpec((NC, _BLK, D), lambda i: (0, i, 0)),
        pl.BlockSpec((NC, _BLK, DEGW), lambda i: (0, i, 0)),
        pl.BlockSpec((1, D), lambda i: (0, 0)),
    ],
    out_specs=pl.BlockSpec((_BLK, D), lambda i: (i, 0)),
    out_shape=jax.ShapeDtypeStruct((N, D), jnp.float32),
)


@jax.jit
def kernel(nodes_features, hyperedge_index, W1, b1, W2, b2):
    row = hyperedge_index[0].astype(jnp.int32)
    col = hyperedge_index[1].astype(jnp.int32)
    npad = NINC_PAD - NINC
    ar = jnp.arange(npad, dtype=jnp.int32)
    pad_g = (ar * 97) % N
    pad_s = N + ar % (NP - N)
    shp_d = (NINC_PAD // DEG_CHUNK, DEG_CHUNK)
    shp_p = (NINC_PAD // PLAIN_CHUNK, PLAIN_CHUNK)
    row_g = jnp.concatenate([row, pad_g])
    row_s = jnp.concatenate([row, pad_s])
    col_g = jnp.concatenate([col, pad_g])
    col_s = jnp.concatenate([col, pad_s])

    z64 = jnp.zeros((DEG_CHUNK, D), jnp.float32)
    z128p = jnp.zeros((PLAIN_CHUNK, D), jnp.float32)
    z18 = jnp.concatenate([jnp.zeros((128, DEGW), jnp.float32),
                           jnp.ones((128, DEGW), jnp.float32)])
    b1r = b1.reshape(1, D)
    b2r = b2.reshape(1, D)

    def _r(p):
        return p.reshape(NC, NP, p.shape[-1])

    xp1 = _mm(nodes_features, W1)
    pe, bp = _scatter_deg(xp1, row_g.reshape(shp_d), col_s.reshape(shp_d),
                          z64, z18)
    ef = _combine_scale(_r(pe), _r(bp))
    pn, dp = _scatter_deg(ef, col_g.reshape(shp_d), row_s.reshape(shp_d),
                          z64, z18)
    xp2 = _combine_relu_mm(_r(pn), _r(dp), b1r, W2)
    (pe2,) = _scatter_plain(xp2, row_g.reshape(shp_p), col_s.reshape(shp_p),
                            z128p, z18)
    ef2 = _combine_scale(_r(pe2), _r(bp))
    (pn2,) = _scatter_plain(ef2, col_g.reshape(shp_p), row_s.reshape(shp_p),
                            z128p, z18)
    return _combine_relu(_r(pn2), _r(dp), b2r)
